# Initial kernel scaffold; baseline (speedup 1.0000x reference)
import jax
import jax.numpy as jnp
from jax.experimental import pallas as pl


def kernel(x, label, mask, edge_index, edge_weight, W1, b1, W2, b2):
    raise NotImplementedError("write your pallas kernel here")


# speedup vs baseline: 7.7592x; 7.7592x over previous
import functools

import jax
import jax.numpy as jnp
from jax import lax
from jax.experimental import pallas as pl
from jax.experimental.pallas import tpu as pltpu
from jax.experimental.pallas import tpu_sc as plsc

NC = 2
NS = 16
NW = NC * NS
LANES = 16


def _make_propagate(n, d, e):
    epw = e // NW
    C = 80
    SCK = 25
    nsc = epw // (C * SCK)
    rps = 640
    n_pad = NS * rps
    ZR = 32
    nz = rps // ZR
    cg_n = d // LANES

    mesh = plsc.VectorSubcoreMesh(
        core_axis_name="c", subcore_axis_name="s", num_cores=NC, num_subcores=NS
    )

    @functools.partial(
        pl.kernel,
        out_type=jax.ShapeDtypeStruct((NC, n_pad, d), jnp.float32),
        mesh=mesh,
        compiler_params=pltpu.CompilerParams(use_tc_tiling_on_sc=(d >= 128)),
        scratch_types=[
            pltpu.VMEM((SCK, C), jnp.int32),
            pltpu.VMEM((SCK, C), jnp.int32),
            pltpu.VMEM((SCK, C), jnp.float32),
            pltpu.VMEM((C, d), jnp.float32),
            pltpu.VMEM((ZR, d), jnp.float32),
            pltpu.VMEM_SHARED((n_pad, d), jnp.float32),
            pltpu.SemaphoreType.DMA,
            pltpu.SemaphoreType.DMA,
        ],
    )
    def prop(z_hbm, src_hbm, dst_hbm, w_hbm, out_hbm,
             src_v, dst_v, w_v, rows_v, zbuf, acc, gsem, ssem):
        cid = lax.axis_index("c")
        sid = lax.axis_index("s")
        wid = cid * NS + sid

        zeros16 = jnp.zeros((LANES,), jnp.float32)

        def zrow(r, carry):
            for cg in range(cg_n):
                zbuf[r, pl.ds(cg * LANES, LANES)] = zeros16
            return carry

        lax.fori_loop(0, ZR, zrow, 0)
        base = sid * rps
        for zi in range(nz):
            pltpu.sync_copy(zbuf, acc.at[pl.ds(base + zi * ZR, ZR)])
        plsc.subcore_barrier()

        def superchunk(j, carry):
            pltpu.sync_copy(src_hbm.at[wid].at[j], src_v)
            pltpu.sync_copy(dst_hbm.at[wid].at[j], dst_v)
            pltpu.sync_copy(w_hbm.at[wid].at[j], w_v)

            def chunk(k, carry2):
                pltpu.async_copy(z_hbm.at[src_v.at[k]], rows_v, gsem).wait()
                for g in range(C // LANES):
                    w_g = w_v[k, pl.ds(g * LANES, LANES)]
                    for i in range(LANES):
                        ee = g * LANES + i
                        w_b = w_g.at[jnp.full((LANES,), i, jnp.int32)].get(
                            mode="promise_in_bounds")
                        for cg in range(cg_n):
                            sl = pl.ds(cg * LANES, LANES)
                            rows_v[ee, sl] = rows_v[ee, sl] * w_b
                pltpu.async_copy(rows_v, acc.at[dst_v.at[k]], ssem,
                                 add=True).wait()
                return carry2

            lax.fori_loop(0, SCK, chunk, 0)
            return carry

        lax.fori_loop(0, nsc, superchunk, 0)
        plsc.subcore_barrier()

        pltpu.sync_copy(acc.at[pl.ds(base, rps)],
                        out_hbm.at[cid].at[pl.ds(base, rps)])

    return prop


def _matmul(x, w):
    n, din = x.shape
    dout = w.shape[1]
    bm = 1000

    def body(x_ref, w_ref, o_ref):
        o_ref[...] = jnp.dot(x_ref[...], w_ref[...],
                             preferred_element_type=jnp.float32)

    return pl.pallas_call(
        body,
        grid=(n // bm,),
        in_specs=[pl.BlockSpec((bm, din), lambda i: (i, 0)),
                  pl.BlockSpec((din, dout), lambda i: (0, 0))],
        out_specs=pl.BlockSpec((bm, dout), lambda i: (i, 0)),
        out_shape=jax.ShapeDtypeStruct((n, dout), jnp.float32),
    )(x, w)


def _combine_relu_matmul(p, b1, w2, n):
    din = p.shape[2]
    dout = w2.shape[1]
    bm = 1000
    b1r = b1.reshape(1, din)

    def body(p_ref, b_ref, w_ref, o_ref):
        h = jnp.maximum(p_ref[0] + p_ref[1] + b_ref[...], 0.0)
        o_ref[...] = jnp.dot(h, w_ref[...], preferred_element_type=jnp.float32)

    return pl.pallas_call(
        body,
        grid=(n // bm,),
        in_specs=[pl.BlockSpec((2, bm, din), lambda i: (0, i, 0)),
                  pl.BlockSpec((1, din), lambda i: (0, 0)),
                  pl.BlockSpec((din, dout), lambda i: (0, 0))],
        out_specs=pl.BlockSpec((bm, dout), lambda i: (i, 0)),
        out_shape=jax.ShapeDtypeStruct((n, dout), jnp.float32),
    )(p, b1r, w2)


def _combine_bias(q, b2, n):
    d = q.shape[2]
    bm = 1000
    b2r = b2.reshape(1, d)

    def body(q_ref, b_ref, o_ref):
        o_ref[...] = q_ref[0] + q_ref[1] + b_ref[...]

    return pl.pallas_call(
        body,
        grid=(n // bm,),
        in_specs=[pl.BlockSpec((2, bm, d), lambda i: (0, i, 0)),
                  pl.BlockSpec((1, d), lambda i: (0, 0))],
        out_specs=pl.BlockSpec((bm, d), lambda i: (i, 0)),
        out_shape=jax.ShapeDtypeStruct((n, d), jnp.float32),
    )(q, b2r)


def kernel(x, label, mask, edge_index, edge_weight, W1, b1, W2, b2):
    n, d_in = x.shape
    e = edge_index.shape[1]
    d_h = W1.shape[1]
    d_out = W2.shape[1]
    epw = e // NW

    C, SCK = 80, 25
    nsc = epw // (C * SCK)
    src = edge_index[0].reshape(NW, nsc, SCK, C)
    dst = edge_index[1].reshape(NW, nsc, SCK, C)
    wts = edge_weight.reshape(NW, nsc, SCK, C)

    h0 = _matmul(x, W1)
    p1 = _make_propagate(n, d_h, e)(h0, src, dst, wts)
    h1 = _combine_relu_matmul(p1, b1, W2, n)
    p2 = _make_propagate(n, d_out, e)(h1, src, dst, wts)
    return _combine_bias(p2, b2, n)
